# baseline (speedup 1.0000x reference)
import jax
import jax.numpy as jnp
from jax.experimental import pallas as pl
from jax.experimental.pallas import tpu as pltpu

INPUT_DIM = 2048
NUM_EXPERTS = 16
TOP_K = 2
NUM_TOKENS = 16384

BLK = 2048
NSTEP = NUM_TOKENS // BLK


def _gating_kernel(x_ref, wt_ref, b_ref, w_out_ref, i_out_ref):
    i = pl.program_id(0)
    logits = jnp.dot(x_ref[...], wt_ref[...], preferred_element_type=jnp.float32)
    logits = logits + b_ref[...]
    m = jnp.max(logits, axis=1, keepdims=True)
    e = jnp.exp(logits - m)
    s = jnp.sum(e, axis=1, keepdims=True)
    lanes = jax.lax.broadcasted_iota(jnp.int32, e.shape, 1)
    v1 = jnp.max(e, axis=1, keepdims=True)
    i1 = jnp.min(jnp.where(e == v1, lanes, NUM_EXPERTS), axis=1, keepdims=True)
    e2 = jnp.where(lanes == i1, -1.0, e)
    v2 = jnp.max(e2, axis=1, keepdims=True)
    i2 = jnp.min(jnp.where(e2 == v2, lanes, NUM_EXPERTS), axis=1, keepdims=True)
    w = jnp.concatenate([v1, v2], axis=1) / s
    idx = jnp.concatenate([i1, i2], axis=1)
    w_out_ref[pl.ds(i * BLK, BLK), :] = w
    i_out_ref[pl.ds(i * BLK, BLK), :] = idx


@jax.jit
def kernel(x, W, b):
    wt = W.T
    b2 = b.reshape(1, NUM_EXPERTS)
    w_out, i_out = pl.pallas_call(
        _gating_kernel,
        grid=(NSTEP,),
        in_specs=[
            pl.BlockSpec((BLK, INPUT_DIM), lambda i: (i, 0)),
            pl.BlockSpec((INPUT_DIM, NUM_EXPERTS), lambda i: (0, 0)),
            pl.BlockSpec((1, NUM_EXPERTS), lambda i: (0, 0)),
        ],
        out_specs=[
            pl.BlockSpec((NUM_TOKENS, TOP_K), lambda i: (0, 0)),
            pl.BlockSpec((NUM_TOKENS, TOP_K), lambda i: (0, 0)),
        ],
        out_shape=[
            jax.ShapeDtypeStruct((NUM_TOKENS, TOP_K), jnp.float32),
            jax.ShapeDtypeStruct((NUM_TOKENS, TOP_K), jnp.int32),
        ],
    )(x, wt, b2)
    return (w_out, i_out)


# speedup vs baseline: 1.0714x; 1.0714x over previous
import jax
import jax.numpy as jnp
from jax.experimental import pallas as pl
from jax.experimental.pallas import tpu as pltpu

INPUT_DIM = 2048
NUM_EXPERTS = 16
TOP_K = 2
NUM_TOKENS = 16384

BLK = 2048
NSTEP = NUM_TOKENS // BLK


def _tdot(a, b):
    return jax.lax.dot_general(
        a, b, (((1,), (1,)), ((), ())), preferred_element_type=jnp.float32
    )


def _gating_kernel(x_ref, whi_ref, wlo_ref, b_ref, w_out_ref, i_out_ref):
    i = pl.program_id(0)
    x = x_ref[...]
    x_hi = x.astype(jnp.bfloat16)
    x_lo = (x - x_hi.astype(jnp.float32)).astype(jnp.bfloat16)
    w_hi = whi_ref[...]
    w_lo = wlo_ref[...]
    logits = _tdot(w_hi, x_hi) + _tdot(w_hi, x_lo) + _tdot(w_lo, x_hi)
    logits = logits + b_ref[...]
    m = jnp.max(logits, axis=0, keepdims=True)
    e = jnp.exp(logits - m)
    s = jnp.sum(e, axis=0, keepdims=True)
    experts = jax.lax.broadcasted_iota(jnp.int32, e.shape, 0)
    v1 = jnp.max(e, axis=0, keepdims=True)
    i1 = jnp.min(jnp.where(e == v1, experts, NUM_EXPERTS), axis=0, keepdims=True)
    e2 = jnp.where(experts == i1, -1.0, e)
    v2 = jnp.max(e2, axis=0, keepdims=True)
    i2 = jnp.min(jnp.where(e2 == v2, experts, NUM_EXPERTS), axis=0, keepdims=True)
    w = jnp.concatenate([v1, v2], axis=0) / s
    idx = jnp.concatenate([i1, i2], axis=0)
    w_out_ref[:, pl.ds(i * BLK, BLK)] = w
    i_out_ref[:, pl.ds(i * BLK, BLK)] = idx


@jax.jit
def kernel(x, W, b):
    w_hi = W.astype(jnp.bfloat16)
    w_lo = (W - w_hi.astype(jnp.float32)).astype(jnp.bfloat16)
    b2 = b.reshape(NUM_EXPERTS, 1)
    w_pack, i_pack = pl.pallas_call(
        _gating_kernel,
        grid=(NSTEP,),
        in_specs=[
            pl.BlockSpec((BLK, INPUT_DIM), lambda i: (i, 0)),
            pl.BlockSpec((NUM_EXPERTS, INPUT_DIM), lambda i: (0, 0)),
            pl.BlockSpec((NUM_EXPERTS, INPUT_DIM), lambda i: (0, 0)),
            pl.BlockSpec((NUM_EXPERTS, 1), lambda i: (0, 0)),
        ],
        out_specs=[
            pl.BlockSpec((TOP_K, NUM_TOKENS), lambda i: (0, 0)),
            pl.BlockSpec((TOP_K, NUM_TOKENS), lambda i: (0, 0)),
        ],
        out_shape=[
            jax.ShapeDtypeStruct((TOP_K, NUM_TOKENS), jnp.float32),
            jax.ShapeDtypeStruct((TOP_K, NUM_TOKENS), jnp.int32),
        ],
    )(x, w_hi, w_lo, b2)
    return (w_pack.T, i_pack.T)
